# baseline (speedup 1.0000x reference)
import jax
import jax.numpy as jnp
from jax import lax
from jax.experimental import pallas as pl
from jax.experimental.pallas import tpu as pltpu
from jax.experimental.pallas import tpu_sc as plsc

N_NODES = 10000
N_EDGES = 320000
D = 128
CHUNK = 64
N_CHUNKS = N_EDGES // CHUNK
GSZ = 4
N_GROUPS = N_CHUNKS // GSZ
NC, NS = 2, 16
NW = NC * NS
BASE_L = N_CHUNKS // NW
BASE_G = BASE_L // GSZ
REM_L = N_CHUNKS - BASE_L * NW
RING = 4
ROWS_PER_TILE = 624
ROWS_TAIL = N_NODES - NS * ROWS_PER_TILE
CNT_W = 16


def _sc_scatter_body(edge_hbm, dest_hbm, sums_out, cnts_out,
                     buf0, buf1, buf2, buf3, ig0, ig1, ones_v, zc_v,
                     sums_sh, cnts_sh,
                     sd0, sd1, sd2, sd3, sg0, sg1,
                     ss0, ss1, ss2, ss3, so0, so1, so2, so3):
    cid = lax.axis_index("c")
    sid = lax.axis_index("s")
    wid = sid * NC + cid

    bufs = (buf0, buf1, buf2, buf3)
    igs = (ig0, ig1)
    sds = (sd0, sd1, sd2, sd3)
    sgs = (sg0, sg1)
    sss = (ss0, ss1, ss2, ss3)
    sos = (so0, so1, so2, so3)

    zeros16 = jnp.zeros((16,), jnp.float32)
    ones16 = jnp.ones((16,), jnp.float32)

    def fill_zero(i, _):
        for k in range(D // 16):
            buf0[i, pl.ds(k * 16, 16)] = zeros16
        zc_v[i] = zeros16
        ones_v[i] = ones16
        return 0

    lax.fori_loop(0, CHUNK, fill_zero, 0)

    def start_data(c, b):
        pltpu.async_copy(edge_hbm.at[pl.ds(c * CHUNK, CHUNK)], bufs[b], sds[b])

    def wait_data(b):
        pltpu.make_async_copy(edge_hbm.at[pl.ds(0, CHUNK)], bufs[b], sds[b]).wait()

    def start_idxg(g, p):
        pltpu.async_copy(dest_hbm.at[g], igs[p], sgs[p])

    def wait_idxg(p):
        pltpu.make_async_copy(dest_hbm.at[0], igs[p], sgs[p]).wait()

    def start_scat(u, p):
        pltpu.async_copy(bufs[u], sums_sh.at[igs[p].at[u]], sss[u], add=True)
        pltpu.async_copy(ones_v, cnts_sh.at[igs[p].at[u]], sos[u], add=True)

    def wait_scat(u, p):
        pltpu.make_async_copy(bufs[u], sums_sh.at[igs[p].at[u]], sss[u]).wait()
        pltpu.make_async_copy(ones_v, cnts_sh.at[igs[p].at[u]], sos[u]).wait()

    start = wid * BASE_L
    startg = wid * BASE_G

    start_idxg(startg, 0)
    start_idxg(startg + 1, 1)
    start_data(start + 1, 1)
    start_data(start + 2, 2)
    start_data(start + 3, 3)

    base = sid * ROWS_PER_TILE
    nz = ROWS_PER_TILE // CHUNK
    zt = ROWS_PER_TILE - nz * CHUNK
    for k in range(nz):
        pltpu.async_copy(buf0, sums_sh.at[pl.ds(base + k * CHUNK, CHUNK)], ss0)
        pltpu.async_copy(zc_v, cnts_sh.at[pl.ds(base + k * CHUNK, CHUNK)], so0)
    pltpu.async_copy(buf0.at[pl.ds(0, zt)],
                     sums_sh.at[pl.ds(base + nz * CHUNK, zt)], ss0)
    pltpu.async_copy(zc_v.at[pl.ds(0, zt)],
                     cnts_sh.at[pl.ds(base + nz * CHUNK, zt)], so0)

    @pl.when(sid == NS - 1)
    def _():
        t0 = NS * ROWS_PER_TILE
        pltpu.async_copy(buf0.at[pl.ds(0, ROWS_TAIL)],
                         sums_sh.at[pl.ds(t0, ROWS_TAIL)], ss0)
        pltpu.async_copy(zc_v.at[pl.ds(0, ROWS_TAIL)],
                         cnts_sh.at[pl.ds(t0, ROWS_TAIL)], so0)

    for k in range(nz):
        pltpu.make_async_copy(buf0, sums_sh.at[pl.ds(base, CHUNK)], ss0).wait()
        pltpu.make_async_copy(zc_v, cnts_sh.at[pl.ds(base, CHUNK)], so0).wait()
    pltpu.make_async_copy(buf0.at[pl.ds(0, zt)],
                          sums_sh.at[pl.ds(base, zt)], ss0).wait()
    pltpu.make_async_copy(zc_v.at[pl.ds(0, zt)],
                          cnts_sh.at[pl.ds(base, zt)], so0).wait()

    @pl.when(sid == NS - 1)
    def _():
        pltpu.make_async_copy(buf0.at[pl.ds(0, ROWS_TAIL)],
                              sums_sh.at[pl.ds(0, ROWS_TAIL)], ss0).wait()
        pltpu.make_async_copy(zc_v.at[pl.ds(0, ROWS_TAIL)],
                              cnts_sh.at[pl.ds(0, ROWS_TAIL)], so0).wait()

    plsc.subcore_barrier()

    start_data(start, 0)

    wait_idxg(0)
    wait_data(0)
    start_scat(0, 0)
    for u in (1, 2, 3):
        wait_data(u)
        start_scat(u, 0)
        wait_scat(u - 1, 0)
        start_data(start + u + 3, u - 1)

    def body(j, _):
        for dG, p in ((1, 1), (2, 0)):
            gg = startg + dG + 2 * j
            t0 = start + GSZ * (dG + 2 * j)
            wait_idxg(p)
            wait_data(0)
            start_scat(0, p)
            wait_scat(3, 1 - p)
            start_data(t0 + 3, 3)
            start_idxg(gg + 1, 1 - p)
            for u in (1, 2, 3):
                wait_data(u)
                start_scat(u, p)
                wait_scat(u - 1, p)
                start_data(t0 + u + 3, u - 1)
        return 0

    lax.fori_loop(0, (BASE_G - 3) // 2, body, 0)

    gg = startg + BASE_G - 2
    t0 = start + GSZ * (BASE_G - 2)
    wait_idxg(1)
    wait_data(0)
    start_scat(0, 1)
    wait_scat(3, 0)
    start_data(t0 + 3, 3)
    start_idxg(gg + 1, 0)
    for u in (1, 2, 3):
        wait_data(u)
        start_scat(u, 1)
        wait_scat(u - 1, 1)
        start_data(t0 + u + 3, u - 1)

    t0 = start + GSZ * (BASE_G - 1)
    wait_idxg(0)
    wait_data(0)
    start_scat(0, 0)
    wait_scat(3, 1)
    start_data(t0 + 3, 3)
    for u in (1, 2, 3):
        wait_data(u)
        start_scat(u, 0)
        wait_scat(u - 1, 0)
    wait_scat(3, 0)

    @pl.when(wid < REM_L)
    def _():
        c = NW * BASE_L + wid
        g = N_GROUPS - 2 + (wid >> 2)
        r = wid & 3
        pltpu.sync_copy(dest_hbm.at[g, r], ig0.at[0])
        pltpu.sync_copy(edge_hbm.at[pl.ds(c * CHUNK, CHUNK)], buf0)
        pltpu.sync_copy(buf0, sums_sh.at[ig0.at[0]], add=True)
        pltpu.sync_copy(ones_v, cnts_sh.at[ig0.at[0]], add=True)

    plsc.subcore_barrier()

    pltpu.async_copy(sums_sh.at[pl.ds(base, ROWS_PER_TILE)],
                     sums_out.at[cid, pl.ds(base, ROWS_PER_TILE)], sd0)
    pltpu.async_copy(cnts_sh.at[pl.ds(base, ROWS_PER_TILE)],
                     cnts_out.at[cid, pl.ds(base, ROWS_PER_TILE)], sd1)

    @pl.when(sid == NS - 1)
    def _():
        t9 = NS * ROWS_PER_TILE
        pltpu.async_copy(sums_sh.at[pl.ds(t9, ROWS_TAIL)],
                         sums_out.at[cid, pl.ds(t9, ROWS_TAIL)], sd0)
        pltpu.async_copy(cnts_sh.at[pl.ds(t9, ROWS_TAIL)],
                         cnts_out.at[cid, pl.ds(t9, ROWS_TAIL)], sd1)

    pltpu.make_async_copy(sums_sh.at[pl.ds(base, ROWS_PER_TILE)],
                          sums_out.at[cid, pl.ds(base, ROWS_PER_TILE)],
                          sd0).wait()
    pltpu.make_async_copy(cnts_sh.at[pl.ds(base, ROWS_PER_TILE)],
                          cnts_out.at[cid, pl.ds(base, ROWS_PER_TILE)],
                          sd1).wait()

    @pl.when(sid == NS - 1)
    def _():
        t9 = NS * ROWS_PER_TILE
        pltpu.make_async_copy(sums_sh.at[pl.ds(t9, ROWS_TAIL)],
                              sums_out.at[cid, pl.ds(t9, ROWS_TAIL)],
                              sd0).wait()
        pltpu.make_async_copy(cnts_sh.at[pl.ds(t9, ROWS_TAIL)],
                              cnts_out.at[cid, pl.ds(t9, ROWS_TAIL)],
                              sd1).wait()


@jax.jit
def _sc_scatter(edge_attr, dest_groups):
    mesh = plsc.VectorSubcoreMesh(core_axis_name="c", subcore_axis_name="s")
    return pl.kernel(
        _sc_scatter_body,
        out_type=[
            jax.ShapeDtypeStruct((NC, N_NODES, D), jnp.float32),
            jax.ShapeDtypeStruct((NC, N_NODES, CNT_W), jnp.float32),
        ],
        mesh=mesh,
        scratch_types=(
            [pltpu.VMEM((CHUNK, D), jnp.float32)] * RING
            + [pltpu.VMEM((GSZ, CHUNK), jnp.int32)] * 2
            + [pltpu.VMEM((CHUNK, CNT_W), jnp.float32)] * 2
            + [pltpu.VMEM_SHARED((N_NODES, D), jnp.float32),
               pltpu.VMEM_SHARED((N_NODES, CNT_W), jnp.float32)]
            + [pltpu.SemaphoreType.DMA] * (RING + 2 + 2 * RING)
        ),
        compiler_params=pltpu.CompilerParams(use_tc_tiling_on_sc=False),
        name="scatter_mean_sc",
    )(edge_attr, dest_groups)


BLK = 2000


def _mlp_body(x_ref, s0_ref, s1_ref, c0_ref, c1_ref,
              w1a_ref, w1b_ref, b1_ref, w2_ref, b2_ref, o_ref):
    cnt = c0_ref[0, :, 0:1] + c1_ref[0, :, 0:1]
    agg = (s0_ref[0] + s1_ref[0]) / jnp.maximum(cnt, 1.0)
    h = (jnp.dot(x_ref[...], w1a_ref[...], preferred_element_type=jnp.float32)
         + jnp.dot(agg, w1b_ref[...], preferred_element_type=jnp.float32)
         + b1_ref[...])
    h = jnp.maximum(h, 0.0)
    o_ref[...] = (jnp.dot(h, w2_ref[...], preferred_element_type=jnp.float32)
                  + b2_ref[...])


@jax.jit
def _mlp(x, sums, cnts, w1a, w1b, b1, w2, b2):
    grid = (N_NODES // BLK,)
    row_spec = pl.BlockSpec((BLK, D), lambda i: (i, 0))
    part_spec = lambda w, c: pl.BlockSpec((1, BLK, w), lambda i, c=c: (c, i, 0))
    full_spec = lambda r, w: pl.BlockSpec((r, w), lambda i: (0, 0))
    return pl.pallas_call(
        _mlp_body,
        grid=grid,
        in_specs=[
            row_spec,
            part_spec(D, 0), part_spec(D, 1),
            part_spec(CNT_W, 0), part_spec(CNT_W, 1),
            full_spec(D, D), full_spec(D, D), full_spec(1, D),
            full_spec(D, D), full_spec(1, D),
        ],
        out_specs=row_spec,
        out_shape=jax.ShapeDtypeStruct((N_NODES, D), jnp.float32),
    )(x, sums, sums, cnts, cnts, w1a, w1b, b1, w2, b2)


def kernel(x, edge_index, edge_attr, W1, b1, W2, b2):
    dest = edge_index[1].astype(jnp.int32).reshape(N_GROUPS, GSZ, CHUNK)
    sums, cnts = _sc_scatter(edge_attr, dest)
    return _mlp(x, sums, cnts,
                W1[:D], W1[D:], b1.reshape(1, D), W2, b2.reshape(1, D))


# speedup vs baseline: 1.0796x; 1.0796x over previous
import jax
import jax.numpy as jnp
from jax import lax
from jax.experimental import pallas as pl
from jax.experimental.pallas import tpu as pltpu
from jax.experimental.pallas import tpu_sc as plsc

N_NODES = 10000
N_EDGES = 320000
D = 128
CHUNK = 64
N_CHUNKS = N_EDGES // CHUNK
NC, NS = 2, 16
NW = NC * NS
BASE_L = N_CHUNKS // NW
REM_L = N_CHUNKS - BASE_L * NW
RING = 4
ROWS_PER_TILE = 624
ROWS_TAIL = N_NODES - NS * ROWS_PER_TILE
CNT_W = 16


def _sc_scatter_body(edge_hbm, ei_hbm, sums_out, cnts_out,
                     buf0, buf1, buf2, buf3, idx0, idx1, idx2, idx3,
                     ones_v, zc_v, sums_sh, cnts_sh,
                     sd0, sd1, sd2, sd3, si0, si1, si2, si3,
                     ss0, ss1, ss2, ss3, so0, so1, so2, so3):
    cid = lax.axis_index("c")
    sid = lax.axis_index("s")
    wid = sid * NC + cid

    bufs = (buf0, buf1, buf2, buf3)
    idxs = (idx0, idx1, idx2, idx3)
    sds = (sd0, sd1, sd2, sd3)
    sis = (si0, si1, si2, si3)
    sss = (ss0, ss1, ss2, ss3)
    sos = (so0, so1, so2, so3)

    zeros16 = jnp.zeros((16,), jnp.float32)
    ones16 = jnp.ones((16,), jnp.float32)

    def fill_zero(i, _):
        for k in range(D // 16):
            buf0[i, pl.ds(k * 16, 16)] = zeros16
        zc_v[i] = zeros16
        ones_v[i] = ones16
        return 0

    lax.fori_loop(0, CHUNK, fill_zero, 0)

    H = CHUNK // 2

    def start_loads(c, b):
        pltpu.async_copy(edge_hbm.at[pl.ds(c * CHUNK, H)],
                         bufs[b].at[pl.ds(0, H)], sds[b])
        pltpu.async_copy(edge_hbm.at[pl.ds(c * CHUNK + H, H)],
                         bufs[b].at[pl.ds(H, H)], sds[b])
        pltpu.async_copy(ei_hbm.at[1, pl.ds(c * CHUNK, CHUNK)], idxs[b], sis[b])

    def wait_loads(b):
        pltpu.make_async_copy(edge_hbm.at[pl.ds(0, H)],
                              bufs[b].at[pl.ds(0, H)], sds[b]).wait()
        pltpu.make_async_copy(edge_hbm.at[pl.ds(0, H)],
                              bufs[b].at[pl.ds(H, H)], sds[b]).wait()
        pltpu.make_async_copy(ei_hbm.at[1, pl.ds(0, CHUNK)], idxs[b], sis[b]).wait()

    def start_scat(b):
        pltpu.async_copy(bufs[b], sums_sh.at[idxs[b]], sss[b], add=True)
        pltpu.async_copy(ones_v, cnts_sh.at[idxs[b]], sos[b], add=True)

    def wait_scat(b):
        pltpu.make_async_copy(bufs[b], sums_sh.at[idxs[b]], sss[b]).wait()
        pltpu.make_async_copy(ones_v, cnts_sh.at[idxs[b]], sos[b]).wait()

    start = wid * BASE_L

    start_loads(start + 1, 1)
    start_loads(start + 2, 2)
    start_loads(start + 3, 3)

    base = sid * ROWS_PER_TILE
    nz = ROWS_PER_TILE // CHUNK
    zt = ROWS_PER_TILE - nz * CHUNK
    for k in range(nz):
        pltpu.async_copy(buf0, sums_sh.at[pl.ds(base + k * CHUNK, CHUNK)], ss0)
        pltpu.async_copy(zc_v, cnts_sh.at[pl.ds(base + k * CHUNK, CHUNK)], so0)
    pltpu.async_copy(buf0.at[pl.ds(0, zt)],
                     sums_sh.at[pl.ds(base + nz * CHUNK, zt)], ss0)
    pltpu.async_copy(zc_v.at[pl.ds(0, zt)],
                     cnts_sh.at[pl.ds(base + nz * CHUNK, zt)], so0)

    @pl.when(sid == NS - 1)
    def _():
        t0 = NS * ROWS_PER_TILE
        pltpu.async_copy(buf0.at[pl.ds(0, ROWS_TAIL)],
                         sums_sh.at[pl.ds(t0, ROWS_TAIL)], ss0)
        pltpu.async_copy(zc_v.at[pl.ds(0, ROWS_TAIL)],
                         cnts_sh.at[pl.ds(t0, ROWS_TAIL)], so0)

    for k in range(nz):
        pltpu.make_async_copy(buf0, sums_sh.at[pl.ds(base, CHUNK)], ss0).wait()
        pltpu.make_async_copy(zc_v, cnts_sh.at[pl.ds(base, CHUNK)], so0).wait()
    pltpu.make_async_copy(buf0.at[pl.ds(0, zt)],
                          sums_sh.at[pl.ds(base, zt)], ss0).wait()
    pltpu.make_async_copy(zc_v.at[pl.ds(0, zt)],
                          cnts_sh.at[pl.ds(base, zt)], so0).wait()

    @pl.when(sid == NS - 1)
    def _():
        pltpu.make_async_copy(buf0.at[pl.ds(0, ROWS_TAIL)],
                              sums_sh.at[pl.ds(0, ROWS_TAIL)], ss0).wait()
        pltpu.make_async_copy(zc_v.at[pl.ds(0, ROWS_TAIL)],
                              cnts_sh.at[pl.ds(0, ROWS_TAIL)], so0).wait()

    plsc.subcore_barrier()

    start_loads(start, 0)

    wait_loads(0)
    start_scat(0)

    def body(j, _):
        for u in range(4):
            b = (1 + u) % RING
            c = start + 1 + 4 * j + u
            wait_loads(b)
            start_scat(b)
            wait_scat(u)
            start_loads(c + 3, u)
        return 0

    lax.fori_loop(0, (BASE_L - 4) // 4, body, 0)

    wait_loads(1)
    start_scat(1)
    wait_scat(0)
    wait_loads(2)
    start_scat(2)
    wait_scat(1)
    wait_loads(3)
    start_scat(3)
    wait_scat(2)
    wait_scat(3)

    @pl.when(wid < REM_L)
    def _():
        c = NW * BASE_L + wid
        pltpu.sync_copy(edge_hbm.at[pl.ds(c * CHUNK, CHUNK)], buf0)
        pltpu.sync_copy(ei_hbm.at[1, pl.ds(c * CHUNK, CHUNK)], idx0)
        pltpu.sync_copy(buf0, sums_sh.at[idx0], add=True)
        pltpu.sync_copy(ones_v, cnts_sh.at[idx0], add=True)

    plsc.subcore_barrier()

    pltpu.async_copy(sums_sh.at[pl.ds(base, ROWS_PER_TILE)],
                     sums_out.at[cid, pl.ds(base, ROWS_PER_TILE)], sd0)
    pltpu.async_copy(cnts_sh.at[pl.ds(base, ROWS_PER_TILE)],
                     cnts_out.at[cid, pl.ds(base, ROWS_PER_TILE)], si0)

    @pl.when(sid == NS - 1)
    def _():
        t0 = NS * ROWS_PER_TILE
        pltpu.async_copy(sums_sh.at[pl.ds(t0, ROWS_TAIL)],
                         sums_out.at[cid, pl.ds(t0, ROWS_TAIL)], sd0)
        pltpu.async_copy(cnts_sh.at[pl.ds(t0, ROWS_TAIL)],
                         cnts_out.at[cid, pl.ds(t0, ROWS_TAIL)], si0)

    pltpu.make_async_copy(sums_sh.at[pl.ds(base, ROWS_PER_TILE)],
                          sums_out.at[cid, pl.ds(base, ROWS_PER_TILE)],
                          sd0).wait()
    pltpu.make_async_copy(cnts_sh.at[pl.ds(base, ROWS_PER_TILE)],
                          cnts_out.at[cid, pl.ds(base, ROWS_PER_TILE)],
                          si0).wait()

    @pl.when(sid == NS - 1)
    def _():
        t0 = NS * ROWS_PER_TILE
        pltpu.make_async_copy(sums_sh.at[pl.ds(t0, ROWS_TAIL)],
                              sums_out.at[cid, pl.ds(t0, ROWS_TAIL)],
                              sd0).wait()
        pltpu.make_async_copy(cnts_sh.at[pl.ds(t0, ROWS_TAIL)],
                              cnts_out.at[cid, pl.ds(t0, ROWS_TAIL)],
                              si0).wait()


@jax.jit
def _sc_scatter(edge_attr, edge_index):
    mesh = plsc.VectorSubcoreMesh(core_axis_name="c", subcore_axis_name="s")
    return pl.kernel(
        _sc_scatter_body,
        out_type=[
            jax.ShapeDtypeStruct((NC, N_NODES, D), jnp.float32),
            jax.ShapeDtypeStruct((NC, N_NODES, CNT_W), jnp.float32),
        ],
        mesh=mesh,
        scratch_types=(
            [pltpu.VMEM((CHUNK, D), jnp.float32)] * RING
            + [pltpu.VMEM((CHUNK,), jnp.int32)] * RING
            + [pltpu.VMEM((CHUNK, CNT_W), jnp.float32)] * 2
            + [pltpu.VMEM_SHARED((N_NODES, D), jnp.float32),
               pltpu.VMEM_SHARED((N_NODES, CNT_W), jnp.float32)]
            + [pltpu.SemaphoreType.DMA] * (4 * RING)
        ),
        compiler_params=pltpu.CompilerParams(use_tc_tiling_on_sc=False),
        name="scatter_mean_sc",
    )(edge_attr, edge_index)


BLK = 2000


def _mlp_body(x_ref, s0_ref, s1_ref, c0_ref, c1_ref,
              w1a_ref, w1b_ref, b1_ref, w2_ref, b2_ref, o_ref):
    cnt = c0_ref[0, :, 0:1] + c1_ref[0, :, 0:1]
    agg = (s0_ref[0] + s1_ref[0]) / jnp.maximum(cnt, 1.0)
    h = (jnp.dot(x_ref[...], w1a_ref[...], preferred_element_type=jnp.float32)
         + jnp.dot(agg, w1b_ref[...], preferred_element_type=jnp.float32)
         + b1_ref[...])
    h = jnp.maximum(h, 0.0)
    o_ref[...] = (jnp.dot(h, w2_ref[...], preferred_element_type=jnp.float32)
                  + b2_ref[...])


@jax.jit
def _mlp(x, sums, cnts, w1a, w1b, b1, w2, b2):
    grid = (N_NODES // BLK,)
    row_spec = pl.BlockSpec((BLK, D), lambda i: (i, 0))
    part_spec = lambda w, c: pl.BlockSpec((1, BLK, w), lambda i, c=c: (c, i, 0))
    full_spec = lambda r, w: pl.BlockSpec((r, w), lambda i: (0, 0))
    return pl.pallas_call(
        _mlp_body,
        grid=grid,
        in_specs=[
            row_spec,
            part_spec(D, 0), part_spec(D, 1),
            part_spec(CNT_W, 0), part_spec(CNT_W, 1),
            full_spec(D, D), full_spec(D, D), full_spec(1, D),
            full_spec(D, D), full_spec(1, D),
        ],
        out_specs=row_spec,
        out_shape=jax.ShapeDtypeStruct((N_NODES, D), jnp.float32),
    )(x, sums, sums, cnts, cnts, w1a, w1b, b1, w2, b2)


def kernel(x, edge_index, edge_attr, W1, b1, W2, b2):
    sums, cnts = _sc_scatter(edge_attr, edge_index.astype(jnp.int32))
    return _mlp(x, sums, cnts,
                W1[:D], W1[D:], b1.reshape(1, D), W2, b2.reshape(1, D))
